# Initial kernel scaffold; baseline (speedup 1.0000x reference)
import jax
import jax.numpy as jnp
from jax.experimental import pallas as pl


def kernel(h, edge_index, edge_attr, params):
    raise NotImplementedError("write your pallas kernel here")


# speedup vs baseline: 3.5396x; 3.5396x over previous
import functools
import math

import jax
import jax.numpy as jnp
from jax.experimental import pallas as pl
from jax.experimental.pallas import tpu as pltpu

_CUTOFF = 10.0
_MAXNB = 32
_NG = 50
_ROWS = 128


def _ssp(x):
    return jax.nn.softplus(x) - math.log(2.0)


def _copy_body(x_ref, o_ref):
    o_ref[...] = x_ref[...]


def _pallas_copy(x, interpret=False):
    return pl.pallas_call(_copy_body,
                          out_shape=jax.ShapeDtypeStruct(x.shape, x.dtype),
                          interpret=interpret)(x)


def _topk_body(pos_blk_ref, pos_t_ref, nbr_ref, negd2_ref, d2_ref, *, n, np_cols):
    b = pl.program_id(0)
    pos_blk = pos_blk_ref[...]
    pos_t = pos_t_ref[...]
    s = jax.lax.dot_general(pos_blk, pos_t, (((1,), (0,)), ((), ())),
                            preferred_element_type=jnp.float32)
    sq_r = jnp.sum(pos_blk * pos_blk, axis=1, keepdims=True)
    sq_c = jnp.sum(pos_t * pos_t, axis=0, keepdims=True)
    d2 = jnp.maximum(sq_r + sq_c - 2.0 * s, 0.0)
    col = jax.lax.broadcasted_iota(jnp.int32, (_ROWS, np_cols), 1)
    rowg = b * _ROWS + jax.lax.broadcasted_iota(jnp.int32, (_ROWS, np_cols), 0)
    inf = jnp.float32(jnp.inf)
    bad = (col == rowg) | (col >= n) | (rowg >= n) | (d2 > _CUTOFF * _CUTOFF)
    d2_ref[...] = jnp.where(bad, inf, d2)

    def cond(c):
        k, cont, _, _ = c
        return (k < _MAXNB) & cont

    def body(c):
        k, cont, nbr, vals = c
        d2c = d2_ref[...]
        m = jnp.min(d2c, axis=1)
        validk = m < inf
        idx = jnp.min(jnp.where(d2c == m[:, None], col, np_cols),
                      axis=1).astype(jnp.int32)
        slot = jax.lax.broadcasted_iota(jnp.int32, (_ROWS, _MAXNB), 1)
        nbr = jnp.where(slot == k, idx[:, None], nbr)
        vals = jnp.where(slot == k,
                         jnp.where(validk, -m, -inf)[:, None], vals)
        d2_ref[...] = jnp.where(col == idx[:, None], inf, d2c)
        return (k + jnp.int32(1), jnp.any(validk), nbr, vals)

    init = (jnp.int32(0), jnp.bool_(True),
            jnp.zeros((_ROWS, _MAXNB), jnp.int32),
            jnp.full((_ROWS, _MAXNB), -inf, jnp.float32))
    _, _, nbr, vals = jax.lax.while_loop(cond, body, init)
    nbr_ref[...] = nbr
    negd2_ref[...] = vals


def _radius_topk(pos, interpret=False):
    n, d = pos.shape
    npr = ((n + _ROWS - 1) // _ROWS) * _ROWS
    np_cols = ((n + 1023) // 1024) * 1024
    pos_pad = jnp.zeros((max(npr, np_cols), d), pos.dtype).at[:n].set(pos)
    kern = functools.partial(_topk_body, n=n, np_cols=np_cols)
    nbr, negd2 = pl.pallas_call(
        kern,
        grid=(npr // _ROWS,),
        in_specs=[
            pl.BlockSpec((_ROWS, d), lambda b: (b, 0)),
            pl.BlockSpec((d, np_cols), lambda b: (0, 0)),
        ],
        out_specs=[
            pl.BlockSpec((_ROWS, _MAXNB), lambda b: (b, 0)),
            pl.BlockSpec((_ROWS, _MAXNB), lambda b: (b, 0)),
        ],
        out_shape=[
            jax.ShapeDtypeStruct((n, _MAXNB), jnp.int32),
            jax.ShapeDtypeStruct((n, _MAXNB), jnp.float32),
        ],
        scratch_shapes=[pltpu.VMEM((_ROWS, np_cols), jnp.float32)],
        interpret=interpret,
    )(pos_pad[:npr], pos_pad[:np_cols].T)
    return nbr, negd2


def _forward(h, params, interpret=False):
    n = h.shape[0]
    hemb = params['emb1'][h[:, 0]] + params['emb1'][h[:, 1]]
    pos = hemb[:, 2:]
    p = jax.lax.stop_gradient(pos)
    nbr, negd2 = _radius_topk(_pallas_copy(p, interpret=interpret),
                              interpret=interpret)
    valid = jnp.isfinite(negd2).reshape(-1)
    dst = jnp.repeat(jnp.arange(n), _MAXNB)
    src = nbr.reshape(-1)
    diff = pos[src] - pos[dst]
    s = jnp.sum(diff * diff, axis=-1)
    ew = jnp.where(s > 0, jnp.sqrt(jnp.where(s > 0, s, 1.0)), 0.0)
    ew = jnp.where(valid, ew, _CUTOFF)
    offset = jnp.linspace(0.0, _CUTOFF, _NG)
    coeff = -0.5 / (offset[1] - offset[0]) ** 2
    ea = jnp.exp(coeff * (ew[:, None] - offset[None, :]) ** 2)
    C = 0.5 * (jnp.cos(ew * math.pi / _CUTOFF) + 1.0)
    x = hemb
    for blk in params['blocks']:
        Wf = (_ssp(ea @ blk['mlp_w1'].T + blk['mlp_b1']) @ blk['mlp_w2'].T
              + blk['mlp_b2']) * C[:, None]
        xl = x @ blk['lin1_w'].T
        agg = jax.ops.segment_sum(xl[src] * Wf, dst, num_segments=n)
        xc = _ssp(agg @ blk['lin2_w'].T + blk['lin2_b'])
        xc = xc @ blk['lin_w'].T + blk['lin_b']
        x = x + xc
        mu = jnp.mean(x, axis=0)
        var = jnp.mean((x - mu) ** 2, axis=0)
        x = (x - mu) / jnp.sqrt(var + 1e-5) * blk['bn_g'] + blk['bn_b']
    out = jnp.sum(x, axis=0, keepdims=True)
    g = _ssp(out @ params['pred_w1'].T + params['pred_b1'])
    pred = g @ params['pred_w2'].T + params['pred_b2']
    return (pred, out, x)


def kernel(h, edge_index, edge_attr, params):
    return _forward(h, params)
